# Initial kernel scaffold; baseline (speedup 1.0000x reference)
import functools

import jax
import jax.numpy as jnp
from jax import lax
from jax.experimental import pallas as pl
from jax.experimental.pallas import tpu as pltpu
from jax.experimental.pallas import tpu_sc as plsc

N = 1000000
D = 32
NC = 2
NS = 16
NW = NC * NS
C = 125
K = N // (NW * C)


def _scatter_kernel(values_hbm, perm_hbm, out_hbm, idx_v, rows_v, sem):
    wid = lax.axis_index("s") * NC + lax.axis_index("c")

    def body(j, carry):
        chunk = wid * K + j
        pltpu.sync_copy(perm_hbm.at[chunk], idx_v)
        pltpu.sync_copy(values_hbm.at[pl.ds(chunk * C, C)], rows_v)
        pltpu.async_copy(rows_v, out_hbm.at[idx_v], sem).wait()
        return carry

    lax.fori_loop(0, K, body, 0)


@jax.jit
def _run(values, perm2d):
    mesh = plsc.VectorSubcoreMesh(core_axis_name="c", subcore_axis_name="s")
    f = functools.partial(
        pl.kernel,
        out_type=jax.ShapeDtypeStruct((N, D), jnp.float32),
        mesh=mesh,
        scratch_types=[
            pltpu.VMEM((C,), jnp.int32),
            pltpu.VMEM((C, D), jnp.float32),
            pltpu.SemaphoreType.DMA,
        ],
    )(_scatter_kernel)
    return f(values, perm2d)


def kernel(values, unbucketize_permute, num_ids_each_rank_to_send,
           num_ids_each_rank_to_receive):
    perm2d = unbucketize_permute.reshape(NW * K, C)
    return _run(values, perm2d)


# speedup vs baseline: 3.9558x; 3.9558x over previous
import functools

import jax
import jax.numpy as jnp
from jax import lax
from jax.experimental import pallas as pl
from jax.experimental.pallas import tpu as pltpu
from jax.experimental.pallas import tpu_sc as plsc

N = 1000000
D = 32
NC = 2
NS = 16
NW = NC * NS
C = 80
M = N // C


def _scatter_kernel(values_hbm, perm_hbm, out_hbm, idx_v, rows_v, sem):
    wid = lax.axis_index("s") * NC + lax.axis_index("c")
    nchunks = (M - 1 - wid) // NW + 1

    def body(j, carry):
        chunk = wid + j * NW
        pltpu.sync_copy(perm_hbm.at[pl.ds(chunk * C, C)], idx_v)
        pltpu.sync_copy(values_hbm.at[pl.ds(chunk * C, C)], rows_v)
        pltpu.async_copy(rows_v, out_hbm.at[idx_v], sem).wait()
        return carry

    lax.fori_loop(0, nchunks, body, 0)


@jax.jit
def _run(values, perm):
    mesh = plsc.VectorSubcoreMesh(core_axis_name="c", subcore_axis_name="s")
    f = functools.partial(
        pl.kernel,
        out_type=jax.ShapeDtypeStruct((N, D), jnp.float32),
        mesh=mesh,
        scratch_types=[
            pltpu.VMEM((C,), jnp.int32),
            pltpu.VMEM((C, D), jnp.float32),
            pltpu.SemaphoreType.DMA,
        ],
        compiler_params=pltpu.CompilerParams(use_tc_tiling_on_sc=False),
    )(_scatter_kernel)
    return f(values, perm)


def kernel(values, unbucketize_permute, num_ids_each_rank_to_send,
           num_ids_each_rank_to_receive):
    return _run(values, unbucketize_permute)


# speedup vs baseline: 5.5034x; 1.3912x over previous
import functools

import jax
import jax.numpy as jnp
from jax import lax
from jax.experimental import pallas as pl
from jax.experimental.pallas import tpu as pltpu
from jax.experimental.pallas import tpu_sc as plsc

N = 1000000
D = 32
NC = 2
NS = 16
NW = NC * NS
C = 80
KB = 20
S = C * KB
G = N // S
NBUF = 2


def _scatter_kernel(values_hbm, perm_hbm, out_hbm, idx_v, rows_v, sem_load,
                    sem_scat):
    wid = lax.axis_index("s") * NC + lax.axis_index("c")
    nt = (G - 1 - wid) // NW + 1

    def start_loads(t, b):
        g = wid + t * NW
        pltpu.async_copy(values_hbm.at[g], rows_v.at[b], sem_load.at[b])
        pltpu.async_copy(perm_hbm.at[g], idx_v.at[b], sem_load.at[b])

    def wait_loads(t, b):
        g = wid + t * NW
        pltpu.make_async_copy(values_hbm.at[g], rows_v.at[b],
                              sem_load.at[b]).wait()
        pltpu.make_async_copy(perm_hbm.at[g], idx_v.at[b],
                              sem_load.at[b]).wait()

    start_loads(0, 0)

    def body(t, carry):
        b = lax.rem(t, NBUF)
        nb = lax.rem(t + 1, NBUF)

        @pl.when(t + 1 < nt)
        def _():
            start_loads(t + 1, nb)

        wait_loads(t, b)
        descs = [
            pltpu.async_copy(rows_v.at[b, j], out_hbm.at[idx_v.at[b, j]],
                             sem_scat.at[b])
            for j in range(KB)
        ]
        for dsc in descs:
            dsc.wait()
        return carry

    lax.fori_loop(0, nt, body, 0)


@jax.jit
def _run(values, perm):
    mesh = plsc.VectorSubcoreMesh(core_axis_name="c", subcore_axis_name="s")
    f = functools.partial(
        pl.kernel,
        out_type=jax.ShapeDtypeStruct((N, D), jnp.float32),
        mesh=mesh,
        scratch_types=[
            pltpu.VMEM((NBUF, KB, C), jnp.int32),
            pltpu.VMEM((NBUF, KB, C, D), jnp.float32),
            pltpu.SemaphoreType.DMA((NBUF,)),
            pltpu.SemaphoreType.DMA((NBUF,)),
        ],
        compiler_params=pltpu.CompilerParams(use_tc_tiling_on_sc=False),
    )(_scatter_kernel)
    vals4 = values.reshape(G, KB, C, D)
    perm3 = perm.reshape(G, KB, C)
    out = f(vals4, perm3)
    return out


def kernel(values, unbucketize_permute, num_ids_each_rank_to_send,
           num_ids_each_rank_to_receive):
    return _run(values, unbucketize_permute)


# speedup vs baseline: 5.5063x; 1.0005x over previous
import functools

import jax
import jax.numpy as jnp
from jax import lax
from jax.experimental import pallas as pl
from jax.experimental.pallas import tpu as pltpu
from jax.experimental.pallas import tpu_sc as plsc

N = 1000000
D = 32
NC = 2
NS = 16
NW = NC * NS
C = 80
KB = 20
S = C * KB
G = N // S
NBUF = 2


def _scatter_kernel(values_hbm, perm_hbm, out_hbm, idx_v, rows_v, sem_load,
                    sem_scat):
    wid = lax.axis_index("s") * NC + lax.axis_index("c")
    nt = (G - 1 - wid) // NW + 1

    def start_loads(t, b):
        g = wid + t * NW
        pltpu.async_copy(values_hbm.at[pl.ds(g * S, S)], rows_v.at[b],
                         sem_load.at[b])
        for j in range(KB):
            pltpu.async_copy(perm_hbm.at[pl.ds(g * S + j * C, C)],
                             idx_v.at[b, j], sem_load.at[b])

    def wait_loads(t, b):
        g = wid + t * NW
        pltpu.make_async_copy(values_hbm.at[pl.ds(g * S, S)], rows_v.at[b],
                              sem_load.at[b]).wait()
        for j in range(KB):
            pltpu.make_async_copy(perm_hbm.at[pl.ds(g * S + j * C, C)],
                                  idx_v.at[b, j], sem_load.at[b]).wait()

    start_loads(0, 0)

    def body(t, carry):
        b = lax.rem(t, NBUF)
        nb = lax.rem(t + 1, NBUF)

        @pl.when(t + 1 < nt)
        def _():
            start_loads(t + 1, nb)

        wait_loads(t, b)
        descs = [
            pltpu.async_copy(rows_v.at[b, pl.ds(j * C, C)],
                             out_hbm.at[idx_v.at[b, j]], sem_scat.at[b])
            for j in range(KB)
        ]
        for dsc in descs:
            dsc.wait()
        return carry

    lax.fori_loop(0, nt, body, 0)


@jax.jit
def _run(values, perm):
    mesh = plsc.VectorSubcoreMesh(core_axis_name="c", subcore_axis_name="s")
    f = functools.partial(
        pl.kernel,
        out_type=jax.ShapeDtypeStruct((N, D), jnp.float32),
        mesh=mesh,
        scratch_types=[
            pltpu.VMEM((NBUF, KB, C), jnp.int32),
            pltpu.VMEM((NBUF, S, D), jnp.float32),
            pltpu.SemaphoreType.DMA((NBUF,)),
            pltpu.SemaphoreType.DMA((NBUF,)),
        ],
        compiler_params=pltpu.CompilerParams(use_tc_tiling_on_sc=False),
    )(_scatter_kernel)
    return f(values, perm)


def kernel(values, unbucketize_permute, num_ids_each_rank_to_send,
           num_ids_each_rank_to_receive):
    return _run(values, unbucketize_permute)


# speedup vs baseline: 5.8778x; 1.0675x over previous
import functools

import jax
import jax.numpy as jnp
from jax import lax
from jax.experimental import pallas as pl
from jax.experimental.pallas import tpu as pltpu
from jax.experimental.pallas import tpu_sc as plsc

N = 1000000
D = 32
NC = 2
NS = 16
NW = NC * NS
C = 80
KB = 5
S = C * KB
G = N // S
NBUF = 2


def _k(values_hbm, perm_hbm, out_hbm, idx_v, buf, sem_load, sem_scat):
    wid = lax.axis_index("s") * NC + lax.axis_index("c")
    nt = (G - 1 - wid) // NW + 1

    def start_loads(t, b):
        g = wid + t * NW
        pltpu.async_copy(values_hbm.at[pl.ds(g * S, S)], buf.at[b],
                         sem_load.at[b])
        for j in range(KB):
            pltpu.async_copy(perm_hbm.at[pl.ds(g * S + j * C, C)],
                             idx_v.at[b, j], sem_load.at[b])

    def wait_loads(t, b):
        g = wid + t * NW
        pltpu.make_async_copy(values_hbm.at[pl.ds(g * S, S)], buf.at[b],
                              sem_load.at[b]).wait()
        for j in range(KB):
            pltpu.make_async_copy(perm_hbm.at[pl.ds(g * S + j * C, C)],
                                  idx_v.at[b, j], sem_load.at[b]).wait()

    start_loads(0, 0)

    def body(t, carry):
        b = lax.rem(t, NBUF)
        nb = lax.rem(t + 1, NBUF)

        @pl.when(t + 1 < nt)
        def _():
            start_loads(t + 1, nb)

        wait_loads(t, b)
        descs = [
            pltpu.async_copy(buf.at[b, pl.ds(j * C, C)],
                             out_hbm.at[idx_v.at[b, j]], sem_scat.at[b])
            for j in range(KB)
        ]
        for dsc in descs:
            dsc.wait()
        return carry

    lax.fori_loop(0, nt, body, 0)


@jax.jit
def _run(values, perm):
    mesh = plsc.VectorSubcoreMesh(core_axis_name="c", subcore_axis_name="s")
    f = functools.partial(
        pl.kernel,
        out_type=jax.ShapeDtypeStruct((N, 128), jnp.float32),
        mesh=mesh,
        scratch_types=[
            pltpu.VMEM((NBUF, KB, C), jnp.int32),
            pltpu.VMEM((NBUF, S, 128), jnp.float32),
            pltpu.SemaphoreType.DMA((NBUF,)),
            pltpu.SemaphoreType.DMA((NBUF,)),
        ],
    )(_k)
    vp = jnp.pad(values, ((0, 0), (0, 128 - D)))
    return f(vp, perm)[:, :D]


def kernel(values, unbucketize_permute, num_ids_each_rank_to_send,
           num_ids_each_rank_to_receive):
    return _run(values, unbucketize_permute)


# speedup vs baseline: 6.5011x; 1.1061x over previous
import functools

import jax
import jax.numpy as jnp
from jax import lax
from jax.experimental import pallas as pl
from jax.experimental.pallas import tpu as pltpu
from jax.experimental.pallas import tpu_sc as plsc

N = 1000000
D = 32
NC = 2
NS = 16
NW = NC * NS
C = 80
KB = 2
S = C * KB
G = N // S
NBUF = 2


def _scatter_kernel(values_hbm, perm_hbm, out_hbm, idx_v, buf_a, buf_b,
                    sem_load, sem_scat):
    wid = lax.axis_index("s") * NC + lax.axis_index("c")
    nt = (G - 1 - wid) // NW + 1

    def start_loads(t, b):
        g = wid + t * NW
        pltpu.async_copy(values_hbm.at[pl.ds(g * S, S)], buf_a.at[b],
                         sem_load.at[b])
        for j in range(KB):
            pltpu.async_copy(perm_hbm.at[pl.ds(g * S + j * C, C)],
                             idx_v.at[b, j], sem_load.at[b])

    def wait_loads(t, b):
        g = wid + t * NW
        pltpu.make_async_copy(values_hbm.at[pl.ds(g * S, S)], buf_a.at[b],
                              sem_load.at[b]).wait()
        for j in range(KB):
            pltpu.make_async_copy(perm_hbm.at[pl.ds(g * S + j * C, C)],
                                  idx_v.at[b, j], sem_load.at[b]).wait()

    start_loads(0, 0)

    def body(t, carry):
        b = lax.rem(t, NBUF)
        nb = lax.rem(t + 1, NBUF)

        @pl.when(t + 1 < nt)
        def _():
            start_loads(t + 1, nb)

        wait_loads(t, b)

        def widen(i, carry2):
            buf_b[b, i, pl.ds(0, 16)] = buf_a[b, i, pl.ds(0, 16)]
            buf_b[b, i, pl.ds(16, 16)] = buf_a[b, i, pl.ds(16, 16)]
            return carry2

        lax.fori_loop(0, S, widen, 0)

        descs = [
            pltpu.async_copy(buf_b.at[b, pl.ds(j * C, C)],
                             out_hbm.at[idx_v.at[b, j]], sem_scat.at[b])
            for j in range(KB)
        ]
        for dsc in descs:
            dsc.wait()
        return carry

    lax.fori_loop(0, nt, body, 0)


@jax.jit
def _run(values, perm):
    mesh = plsc.VectorSubcoreMesh(core_axis_name="c", subcore_axis_name="s")
    f = functools.partial(
        pl.kernel,
        out_type=jax.ShapeDtypeStruct((N, 128), jnp.float32),
        mesh=mesh,
        scratch_types=[
            pltpu.VMEM((NBUF, KB, C), jnp.int32),
            pltpu.VMEM((NBUF, S, D), jnp.float32),
            pltpu.VMEM((NBUF, S, 128), jnp.float32),
            pltpu.SemaphoreType.DMA((NBUF,)),
            pltpu.SemaphoreType.DMA((NBUF,)),
        ],
    )(_scatter_kernel)
    return f(values, perm)[:, :D]


def kernel(values, unbucketize_permute, num_ids_each_rank_to_send,
           num_ids_each_rank_to_receive):
    return _run(values, unbucketize_permute)


# speedup vs baseline: 6.7533x; 1.0388x over previous
import functools

import jax
import jax.numpy as jnp
from jax import lax
from jax.experimental import pallas as pl
from jax.experimental.pallas import tpu as pltpu
from jax.experimental.pallas import tpu_sc as plsc

N = 1000000
D = 32
NC = 2
NS = 16
NW = NC * NS
C = 80
KB = 4
S = C * KB
G = N // S
NBUF = 2


def _scatter_kernel(values_hbm, perm_hbm, out_hbm, idx_v, buf_a, buf_b,
                    sem_load, sem_scat):
    wid = lax.axis_index("s") * NC + lax.axis_index("c")
    nt = (G - 1 - wid) // NW + 1

    def start_loads(t, b):
        g = wid + t * NW
        pltpu.async_copy(values_hbm.at[pl.ds(g * S, S)], buf_a.at[b],
                         sem_load.at[b])
        for j in range(KB):
            pltpu.async_copy(perm_hbm.at[pl.ds(g * S + j * C, C)],
                             idx_v.at[b, j], sem_load.at[b])

    def wait_loads(t, b):
        g = wid + t * NW
        pltpu.make_async_copy(values_hbm.at[pl.ds(g * S, S)], buf_a.at[b],
                              sem_load.at[b]).wait()
        for j in range(KB):
            pltpu.make_async_copy(perm_hbm.at[pl.ds(g * S + j * C, C)],
                                  idx_v.at[b, j], sem_load.at[b]).wait()

    start_loads(0, 0)

    def body(t, carry):
        b = lax.rem(t, NBUF)
        nb = lax.rem(t + 1, NBUF)

        @pl.when(t + 1 < nt)
        def _():
            start_loads(t + 1, nb)

        wait_loads(t, b)

        def widen(i4, carry2):
            for u in range(4):
                i = i4 * 4 + u
                buf_b[i, pl.ds(0, 16)] = buf_a[b, i, pl.ds(0, 16)]
                buf_b[i, pl.ds(16, 16)] = buf_a[b, i, pl.ds(16, 16)]
            return carry2

        lax.fori_loop(0, S // 4, widen, 0)

        descs = [
            pltpu.async_copy(buf_b.at[pl.ds(j * C, C)],
                             out_hbm.at[idx_v.at[b, j]], sem_scat)
            for j in range(KB)
        ]
        for dsc in descs:
            dsc.wait()
        return carry

    lax.fori_loop(0, nt, body, 0)


@jax.jit
def _run(values, perm):
    mesh = plsc.VectorSubcoreMesh(core_axis_name="c", subcore_axis_name="s")
    f = functools.partial(
        pl.kernel,
        out_type=jax.ShapeDtypeStruct((N, 128), jnp.float32),
        mesh=mesh,
        scratch_types=[
            pltpu.VMEM((NBUF, KB, C), jnp.int32),
            pltpu.VMEM((NBUF, S, D), jnp.float32),
            pltpu.VMEM((S, 128), jnp.float32),
            pltpu.SemaphoreType.DMA((NBUF,)),
            pltpu.SemaphoreType.DMA,
        ],
    )(_scatter_kernel)
    return f(values, perm)[:, :D]


def kernel(values, unbucketize_permute, num_ids_each_rank_to_send,
           num_ids_each_rank_to_receive):
    return _run(values, unbucketize_permute)


# speedup vs baseline: 7.1764x; 1.0627x over previous
import functools

import jax
import jax.numpy as jnp
from jax import lax
from jax.experimental import pallas as pl
from jax.experimental.pallas import tpu as pltpu
from jax.experimental.pallas import tpu_sc as plsc

N = 1000000
D = 32
NC = 2
NS = 16
NW = NC * NS
C = 80
KB = 4
S = C * KB
G = N // S
NBUF = 2


def _scatter_kernel(values_hbm, perm_hbm, out_hbm, idx_v, buf_a, buf_b,
                    sem_load, sem_scat):
    wid = lax.axis_index("s") * NC + lax.axis_index("c")
    nt = (G - 1 - wid) // NW + 1

    def start_loads(t, b):
        g = wid + t * NW
        pltpu.async_copy(values_hbm.at[pl.ds(g * S, S)], buf_a.at[b],
                         sem_load.at[b])
        for j in range(KB):
            pltpu.async_copy(perm_hbm.at[pl.ds(g * S + j * C, C)],
                             idx_v.at[b, j], sem_load.at[b])

    def wait_loads(t, b):
        g = wid + t * NW
        pltpu.make_async_copy(values_hbm.at[pl.ds(g * S, S)], buf_a.at[b],
                              sem_load.at[b]).wait()
        for j in range(KB):
            pltpu.make_async_copy(perm_hbm.at[pl.ds(g * S + j * C, C)],
                                  idx_v.at[b, j], sem_load.at[b]).wait()

    start_loads(0, 0)

    def body(t, carry):
        b = lax.rem(t, NBUF)
        nb = lax.rem(t + 1, NBUF)

        @pl.when(t + 1 < nt)
        def _():
            start_loads(t + 1, nb)

        wait_loads(t, b)

        descs = []
        for j in range(KB):
            def widen(i4, carry2, _j=j):
                for u in range(4):
                    i = _j * C + i4 * 4 + u
                    buf_b[i, pl.ds(0, 16)] = buf_a[b, i, pl.ds(0, 16)]
                    buf_b[i, pl.ds(16, 16)] = buf_a[b, i, pl.ds(16, 16)]
                return carry2

            lax.fori_loop(0, C // 4, widen, 0)
            descs.append(
                pltpu.async_copy(buf_b.at[pl.ds(j * C, C)],
                                 out_hbm.at[idx_v.at[b, j]], sem_scat))
        for dsc in descs:
            dsc.wait()
        return carry

    lax.fori_loop(0, nt, body, 0)


@jax.jit
def _run(values, perm):
    mesh = plsc.VectorSubcoreMesh(core_axis_name="c", subcore_axis_name="s")
    f = functools.partial(
        pl.kernel,
        out_type=jax.ShapeDtypeStruct((N, 128), jnp.float32),
        mesh=mesh,
        scratch_types=[
            pltpu.VMEM((NBUF, KB, C), jnp.int32),
            pltpu.VMEM((NBUF, S, D), jnp.float32),
            pltpu.VMEM((S, 128), jnp.float32),
            pltpu.SemaphoreType.DMA((NBUF,)),
            pltpu.SemaphoreType.DMA,
        ],
    )(_scatter_kernel)
    return f(values, perm)[:, :D]


def kernel(values, unbucketize_permute, num_ids_each_rank_to_send,
           num_ids_each_rank_to_receive):
    return _run(values, unbucketize_permute)
